# baseline (speedup 1.0000x reference)
import functools

import jax
import jax.numpy as jnp
from jax import lax
from jax.experimental import pallas as pl
from jax.experimental.pallas import tpu as pltpu

_FEAT = 20000
_PAD = 20480
_ROWS = 160
_LANES = 128
_K = 200
_BATCH = 32
_H1 = 512
_H2 = 256
_H1C = 128
_NH1C = _H1 // _H1C


def _select_body(mask_ref, mrow_ref):
    mv = jax.nn.sigmoid(mask_ref[...])

    one_bits = jnp.int32(0x3F800000)

    def val_step(_, carry):
        lo, hi = carry
        mid = (lo + hi) // 2
        t = lax.bitcast_convert_type(mid, jnp.float32)
        c = jnp.sum((mv > t).astype(jnp.int32))
        ge = c >= _K
        return jnp.where(ge, mid, lo), jnp.where(ge, hi, mid)

    lo, hi = lax.fori_loop(0, 31, val_step, (jnp.int32(0), one_bits))
    thr = lax.bitcast_convert_type(hi, jnp.float32)

    n_gt = jnp.sum((mv > thr).astype(jnp.int32))
    r = _K - n_gt

    tied = mv == thr
    flat_idx = (
        lax.broadcasted_iota(jnp.int32, (_ROWS, _LANES), 0) * _LANES
        + lax.broadcasted_iota(jnp.int32, (_ROWS, _LANES), 1)
    )

    def idx_step(_, carry):
        lo_i, hi_i = carry
        mid = (lo_i + hi_i) // 2
        c = jnp.sum((tied & (flat_idx <= mid)).astype(jnp.int32))
        ge = c >= r
        return jnp.where(ge, lo_i, mid), jnp.where(ge, mid, hi_i)

    lo_i, hi_i = lax.fori_loop(
        0, 15, idx_step, (jnp.int32(-1), jnp.int32(_PAD - 1))
    )
    sel = (mv > thr) | (tied & (flat_idx <= hi_i))
    mrow_ref[...] = jnp.where(sel, mv, 0.0)


def _mlp_body(feat_ref, mrow_ref, w1_ref, b1_ref, w2_ref, b2_ref, w3_ref,
              b3_ref, out_ref, acc_ref):
    i = pl.program_id(0)

    masked = feat_ref[...] * mrow_ref[...].reshape(1, _FEAT)
    part = lax.dot_general(
        masked, w1_ref[...], (((1,), (1,)), ((), ())),
        preferred_element_type=jnp.float32)
    acc_ref[:, pl.ds(i * _H1C, _H1C)] = part

    @pl.when(i == _NH1C - 1)
    def _():
        h1 = jnp.maximum(acc_ref[...] + b1_ref[...].reshape(1, _H1), 0.0)
        h2 = lax.dot_general(h1, w2_ref[...], (((1,), (1,)), ((), ())),
                             preferred_element_type=jnp.float32)
        h2 = jnp.maximum(h2 + b2_ref[...].reshape(1, _H2), 0.0)
        res = lax.dot_general(h2, w3_ref[...], (((1,), (1,)), ((), ())),
                              preferred_element_type=jnp.float32)
        out_ref[...] = res[:, 0:1] + b3_ref[0]


@jax.jit
def _run(feature, mask, W1, b1, W2, b2, W3, b3):
    mask_p = jnp.pad(mask, (0, _PAD - _FEAT), constant_values=-1e30)
    mask_p = mask_p.reshape(_ROWS, _LANES)

    mrow2d = pl.pallas_call(
        _select_body,
        out_shape=jax.ShapeDtypeStruct((_ROWS, _LANES), jnp.float32),
    )(mask_p)

    mask_row = mrow2d.reshape(_PAD)[:_FEAT]

    result = pl.pallas_call(
        _mlp_body,
        grid=(_NH1C,),
        in_specs=[
            pl.BlockSpec((_BATCH, _FEAT), lambda i: (0, 0)),
            pl.BlockSpec((_FEAT,), lambda i: (0,)),
            pl.BlockSpec((_H1C, _FEAT), lambda i: (i, 0)),
            pl.BlockSpec((_H1,), lambda i: (0,)),
            pl.BlockSpec((_H2, _H1), lambda i: (0, 0)),
            pl.BlockSpec((_H2,), lambda i: (0,)),
            pl.BlockSpec((8, _H2), lambda i: (0, 0)),
            pl.BlockSpec(memory_space=pltpu.SMEM),
        ],
        out_specs=pl.BlockSpec((_BATCH, 1), lambda i: (0, 0)),
        out_shape=jax.ShapeDtypeStruct((_BATCH, 1), jnp.float32),
        scratch_shapes=[pltpu.VMEM((_BATCH, _H1), jnp.float32)],
    )(feature, mask_row, W1, b1, W2, b2,
      jnp.pad(W3, ((0, 7), (0, 0))), b3)

    mask_vector = jnp.broadcast_to(mask_row[None, :], (_BATCH, _FEAT))
    return result, mask_vector


def kernel(feature, additional, mask, W1, b1, W2, b2, W3, b3):
    return _run(feature, mask, W1, b1, W2, b2, W3, b3)
